# Initial kernel scaffold; baseline (speedup 1.0000x reference)
import functools

import jax
import jax.numpy as jnp
from jax import lax
from jax.experimental import pallas as pl
from jax.experimental.pallas import tpu as pltpu
from jax.experimental.pallas import tpu_sc as plsc

N = 10000
E = 320000
D = 128
R = 8
B = 4

NC = 2
NS = 16
NW = NC * NS
EW = E // NW
C = 80
NCH = EW // C
SROWS = N // NS

BN = 1000
NB = N // BN


def _h_body(c_ref, x_ref, w_ref, h_ref):
    xb = x_ref[...]
    ys = [jnp.dot(xb, w_ref[b], preferred_element_type=jnp.float32)
          for b in range(B)]
    for r in range(R):
        acc = ys[0] * c_ref[r, 0]
        for b in range(1, B):
            acc = acc + ys[b] * c_ref[r, b]
        h_ref[r] = acc


def _h_call(w_comp, x, weight):
    return pl.pallas_call(
        _h_body,
        grid=(NB,),
        in_specs=[
            pl.BlockSpec(memory_space=pltpu.SMEM),
            pl.BlockSpec((BN, D), lambda i: (i, 0)),
            pl.BlockSpec((B, D, D), lambda i: (0, 0, 0)),
        ],
        out_specs=pl.BlockSpec((R, BN, D), lambda i: (0, i, 0)),
        out_shape=jax.ShapeDtypeStruct((R, N, D), jnp.float32),
    )(w_comp, x, weight)


def _g_body(et_ref, src_ref, g_ref):
    g_ref[...] = et_ref[...] * N + src_ref[...]


def _g_call(et2d, src2d):
    return pl.pallas_call(
        _g_body,
        out_shape=jax.ShapeDtypeStruct(et2d.shape, jnp.int32),
    )(et2d, src2d)


_sc_mesh = plsc.VectorSubcoreMesh(core_axis_name="c", subcore_axis_name="s")


@functools.partial(
    pl.kernel,
    out_type=jax.ShapeDtypeStruct((NC * N, D), jnp.float32),
    mesh=_sc_mesh,
    scratch_types=[
        pltpu.VMEM((NCH, C), jnp.int32),
        pltpu.VMEM((NCH, C), jnp.int32),
        pltpu.VMEM((C, D), jnp.float32),
        pltpu.VMEM_SHARED((N, D), jnp.float32),
        pltpu.SemaphoreType.DMA,
    ],
)
def _sc_scatter(h_hbm, g_hbm, dst_hbm, zeros_hbm, out_hbm,
                gidx_v, dst_v, rows_v, acc, sem):
    cid = lax.axis_index("c")
    sid = lax.axis_index("s")
    wid = sid * NC + cid
    pltpu.sync_copy(g_hbm.at[wid], gidx_v)
    pltpu.sync_copy(dst_hbm.at[wid], dst_v)
    pltpu.sync_copy(zeros_hbm, acc.at[pl.ds(sid * SROWS, SROWS)])
    plsc.subcore_barrier()

    @pl.loop(0, NCH)
    def _chunk(j):
        pltpu.async_copy(h_hbm.at[gidx_v.at[j]], rows_v, sem).wait()
        pltpu.sync_copy(rows_v, acc.at[dst_v.at[j]], add=True)

    plsc.subcore_barrier()
    pltpu.sync_copy(acc.at[pl.ds(sid * SROWS, SROWS)],
                    out_hbm.at[pl.ds(cid * N + sid * SROWS, SROWS)])


def _comb_body(p_ref, b_ref, o_ref):
    o_ref[...] = p_ref[0] + p_ref[1] + b_ref[...]


def _comb_call(p, bias2d):
    return pl.pallas_call(
        _comb_body,
        grid=(NB,),
        in_specs=[
            pl.BlockSpec((NC, BN, D), lambda i: (0, i, 0)),
            pl.BlockSpec((1, D), lambda i: (0, 0)),
        ],
        out_specs=pl.BlockSpec((BN, D), lambda i: (i, 0)),
        out_shape=jax.ShapeDtypeStruct((N, D), jnp.float32),
    )(p, bias2d)


def kernel(x, weight, w_comp, h_bias, edge_index, etypes):
    src = edge_index[0]
    dst = edge_index[1]
    h = _h_call(w_comp, x, weight)
    g2d = _g_call(etypes.reshape(E // D, D), src.reshape(E // D, D))
    partials = _sc_scatter(
        h.reshape(R * N, D),
        g2d.reshape(NW, NCH, C),
        dst.reshape(NW, NCH, C),
        jnp.zeros((SROWS, D), jnp.float32),
    )
    return _comb_call(partials.reshape(NC, N, D), h_bias.reshape(1, D))


# speedup vs baseline: 20.8367x; 20.8367x over previous
import functools

import jax
import jax.numpy as jnp
from jax import lax
from jax.experimental import pallas as pl
from jax.experimental.pallas import tpu as pltpu
from jax.experimental.pallas import tpu_sc as plsc

N = 10000
E = 320000
D = 128
R = 8
B = 4

NC = 2
NS = 16
NW = NC * NS
EW = E // NW
C = 80
NCH = EW // C
NP = 10240
SROWS = NP // NS

BN = 1000
NB = N // BN


def _h_body(c_ref, x_ref, w_ref, h_ref):
    xb = x_ref[...]
    ys = [jnp.dot(xb, w_ref[b], preferred_element_type=jnp.float32)
          for b in range(B)]
    for r in range(R):
        acc = ys[0] * c_ref[r, 0]
        for b in range(1, B):
            acc = acc + ys[b] * c_ref[r, b]
        h_ref[r] = acc


def _h_call(w_comp, x, weight):
    return pl.pallas_call(
        _h_body,
        grid=(NB,),
        in_specs=[
            pl.BlockSpec(memory_space=pltpu.SMEM),
            pl.BlockSpec((BN, D), lambda i: (i, 0)),
            pl.BlockSpec((B, D, D), lambda i: (0, 0, 0)),
        ],
        out_specs=pl.BlockSpec((R, BN, D), lambda i: (0, i, 0)),
        out_shape=jax.ShapeDtypeStruct((R, N, D), jnp.float32),
    )(w_comp, x, weight)


def _g_body(et_ref, src_ref, g_ref):
    g_ref[...] = et_ref[...] * N + src_ref[...]


def _g_call(et2d, src2d):
    return pl.pallas_call(
        _g_body,
        out_shape=jax.ShapeDtypeStruct(et2d.shape, jnp.int32),
    )(et2d, src2d)


_sc_mesh = plsc.VectorSubcoreMesh(core_axis_name="c", subcore_axis_name="s")


@functools.partial(
    pl.kernel,
    out_type=jax.ShapeDtypeStruct((NC * NP, D), jnp.float32),
    mesh=_sc_mesh,
    scratch_types=[
        pltpu.VMEM((NCH, C), jnp.int32),
        pltpu.VMEM((NCH, C), jnp.int32),
        pltpu.VMEM((C, D), jnp.float32),
        pltpu.VMEM_SHARED((NP, D), jnp.float32),
        pltpu.SemaphoreType.DMA,
    ],
)
def _sc_scatter(h_hbm, g_hbm, dst_hbm, zeros_hbm, out_hbm,
                gidx_v, dst_v, rows_v, acc, sem):
    cid = lax.axis_index("c")
    sid = lax.axis_index("s")
    wid = sid * NC + cid
    pltpu.sync_copy(g_hbm.at[wid], gidx_v)
    pltpu.sync_copy(dst_hbm.at[wid], dst_v)
    pltpu.sync_copy(zeros_hbm, acc.at[pl.ds(sid * SROWS, SROWS)])
    plsc.subcore_barrier()

    @pl.loop(0, NCH)
    def _chunk(j):
        pltpu.async_copy(h_hbm.at[gidx_v.at[j]], rows_v, sem).wait()
        pltpu.sync_copy(rows_v, acc.at[dst_v.at[j]], add=True)

    plsc.subcore_barrier()
    pltpu.sync_copy(acc.at[pl.ds(sid * SROWS, SROWS)],
                    out_hbm.at[pl.ds(cid * NP + sid * SROWS, SROWS)])


def _comb_body(p_ref, b_ref, o_ref):
    o_ref[...] = p_ref[0] + p_ref[1] + b_ref[...]


def _comb_call(p, bias2d):
    return pl.pallas_call(
        _comb_body,
        grid=(NB,),
        in_specs=[
            pl.BlockSpec((NC, BN, D), lambda i: (0, i, 0)),
            pl.BlockSpec((1, D), lambda i: (0, 0)),
        ],
        out_specs=pl.BlockSpec((BN, D), lambda i: (i, 0)),
        out_shape=jax.ShapeDtypeStruct((N, D), jnp.float32),
    )(p, bias2d)


def kernel(x, weight, w_comp, h_bias, edge_index, etypes):
    src = edge_index[0]
    dst = edge_index[1]
    h = _h_call(w_comp, x, weight)
    g2d = _g_call(etypes.reshape(E // D, D), src.reshape(E // D, D))
    partials = _sc_scatter(
        h.reshape(R * N, D),
        g2d.reshape(NW, NCH, C),
        dst.reshape(NW, NCH, C),
        jnp.zeros((SROWS, D), jnp.float32),
    )
    return _comb_call(partials.reshape(NC, NP, D), h_bias.reshape(1, D))


# speedup vs baseline: 25.6050x; 1.2288x over previous
import functools

import jax
import jax.numpy as jnp
from jax import lax
from jax.experimental import pallas as pl
from jax.experimental.pallas import tpu as pltpu
from jax.experimental.pallas import tpu_sc as plsc

N = 10000
E = 320000
D = 128
R = 8
B = 4

NC = 2
NS = 16
NW = NC * NS
EW = E // NW
C = 80
NCH = EW // C
SB = 25
NSB = NCH // SB
NP = 10240
SROWS = NP // NS

BN = 1000
NB = N // BN


def _h_body(c_ref, x_ref, w_ref, h_ref):
    xb = x_ref[...]
    ys = [jnp.dot(xb, w_ref[b], preferred_element_type=jnp.float32)
          for b in range(B)]
    for r in range(R):
        acc = ys[0] * c_ref[r, 0]
        for b in range(1, B):
            acc = acc + ys[b] * c_ref[r, b]
        h_ref[r] = acc


def _h_call(w_comp, x, weight):
    return pl.pallas_call(
        _h_body,
        grid=(NB,),
        in_specs=[
            pl.BlockSpec(memory_space=pltpu.SMEM),
            pl.BlockSpec((BN, D), lambda i: (i, 0)),
            pl.BlockSpec((B, D, D), lambda i: (0, 0, 0)),
        ],
        out_specs=pl.BlockSpec((R, BN, D), lambda i: (0, i, 0)),
        out_shape=jax.ShapeDtypeStruct((R, N, D), jnp.float32),
    )(w_comp, x, weight)


def _g_body(et_ref, src_ref, g_ref):
    g_ref[...] = et_ref[...] * N + src_ref[...]


def _g_call(et2d, src2d):
    return pl.pallas_call(
        _g_body,
        out_shape=jax.ShapeDtypeStruct(et2d.shape, jnp.int32),
    )(et2d, src2d)


_sc_mesh = plsc.VectorSubcoreMesh(core_axis_name="c", subcore_axis_name="s")


@functools.partial(
    pl.kernel,
    out_type=jax.ShapeDtypeStruct((NC * NP, D), jnp.float32),
    mesh=_sc_mesh,
    scratch_types=[
        pltpu.VMEM((2, SB, C), jnp.int32),
        pltpu.VMEM((2, SB, C), jnp.int32),
        pltpu.VMEM((2, C, D), jnp.float32),
        pltpu.VMEM_SHARED((NP, D), jnp.float32),
        pltpu.SemaphoreType.DMA,
        pltpu.SemaphoreType.DMA,
        pltpu.SemaphoreType.DMA,
    ],
)
def _sc_scatter(h_hbm, g_hbm, dst_hbm, zeros_hbm, out_hbm,
                gidx_v, dst_v, rows_v, acc, sem0, sem1, semi):
    cid = lax.axis_index("c")
    sid = lax.axis_index("s")
    wid = sid * NC + cid

    def _idx_load(sb, buf, sem):
        pltpu.async_copy(g_hbm.at[wid, sb], gidx_v.at[buf], sem)
        pltpu.async_copy(dst_hbm.at[wid, sb], dst_v.at[buf], sem)

    def _idx_drain(buf, sem):
        pltpu.make_async_copy(g_hbm.at[0, 0], gidx_v.at[buf], sem).wait()
        pltpu.make_async_copy(dst_hbm.at[0, 0], dst_v.at[buf], sem).wait()

    _idx_load(0, 0, semi)
    pltpu.sync_copy(zeros_hbm, acc.at[pl.ds(sid * SROWS, SROWS)])
    plsc.subcore_barrier()

    def _gather(ib, j, buf, sem):
        return pltpu.async_copy(h_hbm.at[gidx_v.at[ib, j]], rows_v.at[buf], sem)

    def _drain(buf, sem):
        pltpu.make_async_copy(h_hbm.at[gidx_v.at[0, 0]], rows_v.at[buf], sem).wait()

    def _scatter(ib, j, buf):
        pltpu.sync_copy(rows_v.at[buf], acc.at[dst_v.at[ib, j]], add=True)

    @pl.loop(0, NSB)
    def _superblock(sb):
        ib = lax.rem(sb, 2)
        _idx_drain(ib, semi)

        @pl.when(sb < NSB - 1)
        def _prefetch():
            _idx_load(sb + 1, 1 - ib, semi)

        _gather(ib, 0, 0, sem0)

        @pl.loop(0, (SB - 1) // 2)
        def _pair(k):
            j = 1 + 2 * k
            _drain(0, sem0)
            _gather(ib, j, 1, sem1)
            _scatter(ib, j - 1, 0)
            _drain(1, sem1)
            _gather(ib, j + 1, 0, sem0)
            _scatter(ib, j, 1)

        _drain(0, sem0)
        _scatter(ib, SB - 1, 0)

    plsc.subcore_barrier()
    pltpu.sync_copy(acc.at[pl.ds(sid * SROWS, SROWS)],
                    out_hbm.at[pl.ds(cid * NP + sid * SROWS, SROWS)])


def _comb_body(p_ref, b_ref, o_ref):
    o_ref[...] = p_ref[0] + p_ref[1] + b_ref[...]


def _comb_call(p, bias2d):
    return pl.pallas_call(
        _comb_body,
        grid=(NB,),
        in_specs=[
            pl.BlockSpec((NC, BN, D), lambda i: (0, i, 0)),
            pl.BlockSpec((1, D), lambda i: (0, 0)),
        ],
        out_specs=pl.BlockSpec((BN, D), lambda i: (i, 0)),
        out_shape=jax.ShapeDtypeStruct((N, D), jnp.float32),
    )(p, bias2d)


def kernel(x, weight, w_comp, h_bias, edge_index, etypes):
    src = edge_index[0]
    dst = edge_index[1]
    h = _h_call(w_comp, x, weight)
    g2d = _g_call(etypes.reshape(E // D, D), src.reshape(E // D, D))
    partials = _sc_scatter(
        h.reshape(R * N, D),
        g2d.reshape(NW, NSB, SB, C),
        dst.reshape(NW, NSB, SB, C),
        jnp.zeros((SROWS, D), jnp.float32),
    )
    return _comb_call(partials.reshape(NC, NP, D), h_bias.reshape(1, D))
